# Initial kernel scaffold; baseline (speedup 1.0000x reference)
import jax
import jax.numpy as jnp
from jax.experimental import pallas as pl


def kernel(h, edge_index, e, emb_h, Wa, Wg, W1, b1, W2, b2):
    raise NotImplementedError("write your pallas kernel here")


# speedup vs baseline: 6.2969x; 6.2969x over previous
import functools

import jax
import jax.numpy as jnp
from jax import lax
from jax.experimental import pallas as pl
from jax.experimental.pallas import tpu as pltpu
from jax.experimental.pallas import tpu_sc as plsc

N = 10000
E = 320000
HID = 128
HHALF = HID // 2
IN_DIM = 64
NCLS = 8
NLAYER = 4

NCORES = 2
NSUB = 16
NTILES = NCORES * NSUB
EPT = E // NTILES
CHUNK = 80
NCH = EPT // CHUNK
NPAD = 10240
ROWS_PT = NPAD // NSUB
WCH = 128

_SC_PARAMS = pltpu.CompilerParams(use_tc_tiling_on_sc=False)


def _sc_init_body(h3, emb_st, dsts, x0, degp, hbuf, rows, idxd, ones_v,
                  degbuf, degacc, semg, sems):
    c = lax.axis_index("c")
    s = lax.axis_index("s")

    pltpu.sync_copy(dsts.at[c, s], idxd)

    pltpu.sync_copy(h3.at[s], hbuf)
    cps = [pltpu.async_copy(emb_st.at[c].at[hbuf.at[k]], rows.at[k], semg)
           for k in range(5)]

    @pl.loop(0, CHUNK)
    def _(i):
        ones_v[i, :] = jnp.ones((16,), jnp.float32)

    @pl.loop(0, ROWS_PT)
    def _(i):
        degbuf[i, :] = jnp.zeros((16,), jnp.float32)

    pltpu.sync_copy(degbuf, degacc.at[pl.ds(s * ROWS_PT, ROWS_PT)])
    plsc.subcore_barrier()

    for cp in cps:
        cp.wait()
    pltpu.sync_copy(rows, x0.at[c, s])

    @pl.loop(0, NCH // 5)
    def _(g):
        ds_ = [pltpu.async_copy(ones_v, degacc.at[idxd.at[5 * g + i]], sems,
                                add=True)
               for i in range(5)]
        for d in ds_:
            d.wait()

    plsc.subcore_barrier()

    pltpu.sync_copy(degacc.at[pl.ds(s * ROWS_PT, ROWS_PT)], degbuf)
    pltpu.sync_copy(degbuf, degp.at[c].at[pl.ds(s * ROWS_PT, ROWS_PT)])


@functools.lru_cache(maxsize=None)
def _make_sc_init():
    mesh = plsc.VectorSubcoreMesh(core_axis_name="c", subcore_axis_name="s",
                                  num_cores=NCORES, num_subcores=NSUB)
    return pl.kernel(
        _sc_init_body,
        out_type=(
            jax.ShapeDtypeStruct((NCORES, NSUB, 5, WCH, HHALF), jnp.float32),
            jax.ShapeDtypeStruct((NCORES, NPAD, 16), jnp.float32),
        ),
        mesh=mesh,
        compiler_params=_SC_PARAMS,
        scratch_types=[
            pltpu.VMEM((5, WCH), jnp.int32),
            pltpu.VMEM((5, WCH, HHALF), jnp.float32),
            pltpu.VMEM((NCH, CHUNK), jnp.int32),
            pltpu.VMEM((CHUNK, 16), jnp.float32),
            pltpu.VMEM((ROWS_PT, 16), jnp.float32),
            pltpu.VMEM_SHARED((NPAD, 16), jnp.float32),
            pltpu.SemaphoreType.DMA,
            pltpu.SemaphoreType.DMA,
        ],
    )


def _sc_spmm_body(x_st, srcs, dsts, parts, idxs, idxd, rows_a, rows_b, zbuf,
                  acc, sem_a, sem_b):
    c = lax.axis_index("c")
    s = lax.axis_index("s")

    pltpu.sync_copy(srcs.at[c, s], idxs)
    pltpu.sync_copy(dsts.at[c, s], idxd)

    @pl.loop(0, WCH * (HHALF // 16))
    def _(i):
        zbuf[i // (HHALF // 16), pl.ds((i % (HHALF // 16)) * 16, 16)] = (
            jnp.zeros((16,), jnp.float32))

    for hh in range(2):
        for k in range(ROWS_PT // WCH):
            pltpu.sync_copy(zbuf, acc.at[pl.ds(s * ROWS_PT + k * WCH, WCH)])
        plsc.subcore_barrier()

        @pl.loop(0, (NCH - 1) // 2)
        def _(j):
            cp_a = pltpu.async_copy(x_st.at[hh].at[idxs.at[2 * j]], rows_a,
                                    sem_a)
            cp_b = pltpu.async_copy(x_st.at[hh].at[idxs.at[2 * j + 1]],
                                    rows_b, sem_b)
            cp_a.wait()
            pltpu.sync_copy(rows_a, acc.at[idxd.at[2 * j]], add=True)
            cp_b.wait()
            pltpu.sync_copy(rows_b, acc.at[idxd.at[2 * j + 1]], add=True)

        cp = pltpu.async_copy(x_st.at[hh].at[idxs.at[NCH - 1]], rows_a, sem_a)
        cp.wait()
        pltpu.sync_copy(rows_a, acc.at[idxd.at[NCH - 1]], add=True)

        plsc.subcore_barrier()

        for k in range(ROWS_PT // WCH):
            base = s * ROWS_PT + k * WCH
            pltpu.sync_copy(acc.at[pl.ds(base, WCH)], zbuf)
            pltpu.sync_copy(zbuf, parts.at[hh, c].at[pl.ds(base, WCH)])
        @pl.loop(0, WCH * (HHALF // 16))
        def _(i):
            zbuf[i // (HHALF // 16), pl.ds((i % (HHALF // 16)) * 16, 16)] = (
                jnp.zeros((16,), jnp.float32))
        plsc.subcore_barrier()


@functools.lru_cache(maxsize=None)
def _make_sc_spmm():
    mesh = plsc.VectorSubcoreMesh(core_axis_name="c", subcore_axis_name="s",
                                  num_cores=NCORES, num_subcores=NSUB)
    return pl.kernel(
        _sc_spmm_body,
        out_type=jax.ShapeDtypeStruct((2, NCORES, NPAD, HHALF), jnp.float32),
        mesh=mesh,
        compiler_params=_SC_PARAMS,
        scratch_types=[
            pltpu.VMEM((NCH, CHUNK), jnp.int32),
            pltpu.VMEM((NCH, CHUNK), jnp.int32),
            pltpu.VMEM((CHUNK, HHALF), jnp.float32),
            pltpu.VMEM((CHUNK, HHALF), jnp.float32),
            pltpu.VMEM((WCH, HHALF), jnp.float32),
            pltpu.VMEM_SHARED((NPAD, HHALF), jnp.float32),
            pltpu.SemaphoreType.DMA,
            pltpu.SemaphoreType.DMA,
        ],
    )


BLK = 1000


def _agg_block(parts_ref, degp_ref):
    deg = degp_ref[0, :, 0] + degp_ref[1, :, 0]
    inv = 1.0 / jnp.maximum(deg, 1.0)
    agg = jnp.concatenate(
        [parts_ref[0, 0] + parts_ref[0, 1], parts_ref[1, 0] + parts_ref[1, 1]],
        axis=1)
    return agg * inv[:, None]


def _tc_update_body(parts_ref, degp_ref, x_ref, wg_ref, wa_ref, out_ref):
    agg = _agg_block(parts_ref, degp_ref)
    g = jnp.dot(agg, wg_ref[...], preferred_element_type=jnp.float32)
    z = 1.0 / (1.0 + jnp.exp(-g))
    xb = jnp.concatenate([x_ref[0], x_ref[1]], axis=1)
    xn = xb + z * jnp.dot(agg, wa_ref[...], preferred_element_type=jnp.float32)
    out_ref[0] = xn[:, :HHALF]
    out_ref[1] = xn[:, HHALF:]


_tc_update = pl.pallas_call(
    _tc_update_body,
    out_shape=jax.ShapeDtypeStruct((2, NPAD, HHALF), jnp.float32),
    grid=(N // BLK,),
    in_specs=[
        pl.BlockSpec((2, NCORES, BLK, HHALF), lambda i: (0, 0, i, 0)),
        pl.BlockSpec((NCORES, BLK, 16), lambda i: (0, i, 0)),
        pl.BlockSpec((2, BLK, HHALF), lambda i: (0, i, 0)),
        pl.BlockSpec((HID, HID), lambda i: (0, 0)),
        pl.BlockSpec((HID, HID), lambda i: (0, 0)),
    ],
    out_specs=pl.BlockSpec((2, BLK, HHALF), lambda i: (0, i, 0)),
)


def _tc_readout_body(x_ref, w1_ref, b1_ref, w2_ref, b2_ref, out_ref):
    xb = jnp.concatenate([x_ref[0], x_ref[1]], axis=1)
    mid = jnp.maximum(
        jnp.dot(xb, w1_ref[...], preferred_element_type=jnp.float32)
        + b1_ref[...], 0.0)
    out_ref[...] = jnp.dot(
        mid, w2_ref[...], preferred_element_type=jnp.float32) + b2_ref[...]


_tc_readout = pl.pallas_call(
    _tc_readout_body,
    out_shape=jax.ShapeDtypeStruct((N, NCLS), jnp.float32),
    grid=(N // BLK,),
    in_specs=[
        pl.BlockSpec((2, BLK, HHALF), lambda i: (0, i, 0)),
        pl.BlockSpec((HID, HID // 2), lambda i: (0, 0)),
        pl.BlockSpec((HID // 2,), lambda i: (0,)),
        pl.BlockSpec((HID // 2, NCLS), lambda i: (0, 0)),
        pl.BlockSpec((NCLS,), lambda i: (0,)),
    ],
    out_specs=pl.BlockSpec((BLK, NCLS), lambda i: (i, 0)),
)


def kernel(h, edge_index, e, emb_h, Wa, Wg, W1, b1, W2, b2):
    del e
    h_pad = jnp.concatenate(
        [h.astype(jnp.int32), jnp.zeros((NPAD - N,), jnp.int32)])
    h3 = h_pad.reshape(NSUB, 5, WCH)
    emb_st = jnp.stack([emb_h[:, :HHALF], emb_h[:, HHALF:]])
    srcs = edge_index[0].astype(jnp.int32).reshape(NCORES, NSUB, NCH, CHUNK)
    dsts = edge_index[1].astype(jnp.int32).reshape(NCORES, NSUB, NCH, CHUNK)

    sc_init = _make_sc_init()
    sc_spmm = _make_sc_spmm()
    x0_t, degp = sc_init(h3, emb_st, dsts)
    x_st = x0_t.reshape(NCORES, NPAD, HHALF)

    def layer(l, x_st):
        parts = sc_spmm(x_st, srcs, dsts)
        return _tc_update(parts, degp, x_st, Wg[l], Wa[l])

    x_st = lax.fori_loop(0, NLAYER, layer, x_st)
    return _tc_readout(x_st, W1, b1, W2, b2)


# speedup vs baseline: 7.6860x; 1.2206x over previous
import functools

import jax
import jax.numpy as jnp
from jax import lax
from jax.experimental import pallas as pl
from jax.experimental.pallas import tpu as pltpu
from jax.experimental.pallas import tpu_sc as plsc

N = 10000
E = 320000
HID = 128
HHALF = HID // 2
IN_DIM = 64
NCLS = 8
NLAYER = 4

NCORES = 2
NSUB = 16
NTILES = NCORES * NSUB
EPT = E // NTILES
CHUNK = 80
NCH = EPT // CHUNK
NPAD = 10240
ROWS_PT = NPAD // NSUB
WCH = 128

_SC_PARAMS = pltpu.CompilerParams(use_tc_tiling_on_sc=False)


def _sc_init_body(h3, emb_st, dsts, x0, degp, hbuf, rows, idxd, ones_v,
                  degbuf, degacc, semg, sems):
    c = lax.axis_index("c")
    s = lax.axis_index("s")

    pltpu.sync_copy(dsts.at[c, s], idxd)

    pltpu.sync_copy(h3.at[s], hbuf)
    cps = [pltpu.async_copy(emb_st.at[c].at[hbuf.at[k]], rows.at[k], semg)
           for k in range(5)]

    @pl.loop(0, CHUNK)
    def _(i):
        ones_v[i, :] = jnp.ones((16,), jnp.float32)

    @pl.loop(0, ROWS_PT)
    def _(i):
        degbuf[i, :] = jnp.zeros((16,), jnp.float32)

    pltpu.sync_copy(degbuf, degacc.at[pl.ds(s * ROWS_PT, ROWS_PT)])
    plsc.subcore_barrier()

    for cp in cps:
        cp.wait()
    pltpu.sync_copy(rows, x0.at[c, s])

    @pl.loop(0, NCH // 5)
    def _(g):
        ds_ = [pltpu.async_copy(ones_v, degacc.at[idxd.at[5 * g + i]], sems,
                                add=True)
               for i in range(5)]
        for d in ds_:
            d.wait()

    plsc.subcore_barrier()

    pltpu.sync_copy(degacc.at[pl.ds(s * ROWS_PT, ROWS_PT)], degbuf)
    pltpu.sync_copy(degbuf, degp.at[c].at[pl.ds(s * ROWS_PT, ROWS_PT)])


@functools.lru_cache(maxsize=None)
def _make_sc_init():
    mesh = plsc.VectorSubcoreMesh(core_axis_name="c", subcore_axis_name="s",
                                  num_cores=NCORES, num_subcores=NSUB)
    return pl.kernel(
        _sc_init_body,
        out_type=(
            jax.ShapeDtypeStruct((NCORES, NSUB, 5, WCH, HHALF), jnp.float32),
            jax.ShapeDtypeStruct((NCORES, NPAD, 16), jnp.float32),
        ),
        mesh=mesh,
        compiler_params=_SC_PARAMS,
        scratch_types=[
            pltpu.VMEM((5, WCH), jnp.int32),
            pltpu.VMEM((5, WCH, HHALF), jnp.float32),
            pltpu.VMEM((NCH, CHUNK), jnp.int32),
            pltpu.VMEM((CHUNK, 16), jnp.float32),
            pltpu.VMEM((ROWS_PT, 16), jnp.float32),
            pltpu.VMEM_SHARED((NPAD, 16), jnp.float32),
            pltpu.SemaphoreType.DMA,
            pltpu.SemaphoreType.DMA,
        ],
    )


NBUF = 5


def _sc_spmm_body(x_st, srcs, dsts, parts, idxs, idxd, rows, zbuf,
                  acc, semg, sems):
    c = lax.axis_index("c")
    s = lax.axis_index("s")

    pltpu.sync_copy(srcs.at[c, s], idxs)
    pltpu.sync_copy(dsts.at[c, s], idxd)

    @pl.loop(0, WCH * (HHALF // 16))
    def _(i):
        zbuf[i // (HHALF // 16), pl.ds((i % (HHALF // 16)) * 16, 16)] = (
            jnp.zeros((16,), jnp.float32))

    for hh in range(2):
        for k in range(ROWS_PT // WCH):
            pltpu.sync_copy(zbuf, acc.at[pl.ds(s * ROWS_PT + k * WCH, WCH)])
        plsc.subcore_barrier()

        @pl.loop(0, NCH // NBUF)
        def _(r):
            cps = [pltpu.async_copy(x_st.at[hh].at[idxs.at[NBUF * r + b]],
                                    rows.at[b], semg[b])
                   for b in range(NBUF)]
            scs = []
            for b in range(NBUF):
                cps[b].wait()
                scs.append(pltpu.async_copy(
                    rows.at[b], acc.at[idxd.at[NBUF * r + b]], sems[b],
                    add=True))
            for sc in scs:
                sc.wait()

        plsc.subcore_barrier()

        for k in range(ROWS_PT // WCH):
            base = s * ROWS_PT + k * WCH
            pltpu.sync_copy(acc.at[pl.ds(base, WCH)], zbuf)
            pltpu.sync_copy(zbuf, parts.at[hh, c].at[pl.ds(base, WCH)])
        @pl.loop(0, WCH * (HHALF // 16))
        def _(i):
            zbuf[i // (HHALF // 16), pl.ds((i % (HHALF // 16)) * 16, 16)] = (
                jnp.zeros((16,), jnp.float32))
        plsc.subcore_barrier()


@functools.lru_cache(maxsize=None)
def _make_sc_spmm():
    mesh = plsc.VectorSubcoreMesh(core_axis_name="c", subcore_axis_name="s",
                                  num_cores=NCORES, num_subcores=NSUB)
    return pl.kernel(
        _sc_spmm_body,
        out_type=jax.ShapeDtypeStruct((2, NCORES, NPAD, HHALF), jnp.float32),
        mesh=mesh,
        compiler_params=_SC_PARAMS,
        scratch_types=[
            pltpu.VMEM((NCH, CHUNK), jnp.int32),
            pltpu.VMEM((NCH, CHUNK), jnp.int32),
            pltpu.VMEM((NBUF, CHUNK, HHALF), jnp.float32),
            pltpu.VMEM((WCH, HHALF), jnp.float32),
            pltpu.VMEM_SHARED((NPAD, HHALF), jnp.float32),
            [pltpu.SemaphoreType.DMA] * NBUF,
            [pltpu.SemaphoreType.DMA] * NBUF,
        ],
    )


BLK = 1000


def _agg_block(parts_ref, degp_ref):
    deg = degp_ref[0, :, 0] + degp_ref[1, :, 0]
    inv = 1.0 / jnp.maximum(deg, 1.0)
    agg = jnp.concatenate(
        [parts_ref[0, 0] + parts_ref[0, 1], parts_ref[1, 0] + parts_ref[1, 1]],
        axis=1)
    return agg * inv[:, None]


def _tc_update_body(parts_ref, degp_ref, x_ref, wg_ref, wa_ref, out_ref):
    agg = _agg_block(parts_ref, degp_ref)
    g = jnp.dot(agg, wg_ref[...], preferred_element_type=jnp.float32)
    z = 1.0 / (1.0 + jnp.exp(-g))
    xb = jnp.concatenate([x_ref[0], x_ref[1]], axis=1)
    xn = xb + z * jnp.dot(agg, wa_ref[...], preferred_element_type=jnp.float32)
    out_ref[0] = xn[:, :HHALF]
    out_ref[1] = xn[:, HHALF:]


_tc_update = pl.pallas_call(
    _tc_update_body,
    out_shape=jax.ShapeDtypeStruct((2, NPAD, HHALF), jnp.float32),
    grid=(N // BLK,),
    in_specs=[
        pl.BlockSpec((2, NCORES, BLK, HHALF), lambda i: (0, 0, i, 0)),
        pl.BlockSpec((NCORES, BLK, 16), lambda i: (0, i, 0)),
        pl.BlockSpec((2, BLK, HHALF), lambda i: (0, i, 0)),
        pl.BlockSpec((HID, HID), lambda i: (0, 0)),
        pl.BlockSpec((HID, HID), lambda i: (0, 0)),
    ],
    out_specs=pl.BlockSpec((2, BLK, HHALF), lambda i: (0, i, 0)),
)


def _tc_readout_body(x_ref, w1_ref, b1_ref, w2_ref, b2_ref, out_ref):
    xb = jnp.concatenate([x_ref[0], x_ref[1]], axis=1)
    mid = jnp.maximum(
        jnp.dot(xb, w1_ref[...], preferred_element_type=jnp.float32)
        + b1_ref[...], 0.0)
    out_ref[...] = jnp.dot(
        mid, w2_ref[...], preferred_element_type=jnp.float32) + b2_ref[...]


_tc_readout = pl.pallas_call(
    _tc_readout_body,
    out_shape=jax.ShapeDtypeStruct((N, NCLS), jnp.float32),
    grid=(N // BLK,),
    in_specs=[
        pl.BlockSpec((2, BLK, HHALF), lambda i: (0, i, 0)),
        pl.BlockSpec((HID, HID // 2), lambda i: (0, 0)),
        pl.BlockSpec((HID // 2,), lambda i: (0,)),
        pl.BlockSpec((HID // 2, NCLS), lambda i: (0, 0)),
        pl.BlockSpec((NCLS,), lambda i: (0,)),
    ],
    out_specs=pl.BlockSpec((BLK, NCLS), lambda i: (i, 0)),
)


def kernel(h, edge_index, e, emb_h, Wa, Wg, W1, b1, W2, b2):
    del e
    h_pad = jnp.concatenate(
        [h.astype(jnp.int32), jnp.zeros((NPAD - N,), jnp.int32)])
    h3 = h_pad.reshape(NSUB, 5, WCH)
    emb_st = jnp.stack([emb_h[:, :HHALF], emb_h[:, HHALF:]])
    srcs = edge_index[0].astype(jnp.int32).reshape(NCORES, NSUB, NCH, CHUNK)
    dsts = edge_index[1].astype(jnp.int32).reshape(NCORES, NSUB, NCH, CHUNK)

    sc_init = _make_sc_init()
    sc_spmm = _make_sc_spmm()
    x0_t, degp = sc_init(h3, emb_st, dsts)
    x_st = x0_t.reshape(NCORES, NPAD, HHALF)

    def layer(l, x_st):
        parts = sc_spmm(x_st, srcs, dsts)
        return _tc_update(parts, degp, x_st, Wg[l], Wa[l])

    x_st = lax.fori_loop(0, NLAYER, layer, x_st)
    return _tc_readout(x_st, W1, b1, W2, b2)


# speedup vs baseline: 8.6573x; 1.1264x over previous
import functools

import jax
import jax.numpy as jnp
from jax import lax
from jax.experimental import pallas as pl
from jax.experimental.pallas import tpu as pltpu
from jax.experimental.pallas import tpu_sc as plsc

N = 10000
E = 320000
HID = 128
HHALF = HID // 2
IN_DIM = 64
NCLS = 8
NLAYER = 4

NCORES = 2
NSUB = 16
EPT = E // NSUB
CHUNK = 125
NCHT = EPT // CHUNK
NBUF = 5
RND = NCHT // NBUF
DEGC = NCHT // NCORES
NPAD = 10240
ROWS_PT = NPAD // NSUB
WCH = 128

_SC_PARAMS = pltpu.CompilerParams(use_tc_tiling_on_sc=False)


def _sc_init_body(h3, emb_st, dsts, x0, degp, hbuf, rows, idxd, ones_v,
                  degbuf, degacc, semg, sems):
    c = lax.axis_index("c")
    s = lax.axis_index("s")

    pltpu.sync_copy(dsts.at[s], idxd)

    pltpu.sync_copy(h3.at[s], hbuf)
    cps = [pltpu.async_copy(emb_st.at[c].at[hbuf.at[k]], rows.at[k], semg)
           for k in range(5)]

    @pl.loop(0, CHUNK)
    def _(i):
        ones_v[i, :] = jnp.ones((16,), jnp.float32)

    @pl.loop(0, ROWS_PT)
    def _(i):
        degbuf[i, :] = jnp.zeros((16,), jnp.float32)

    pltpu.sync_copy(degbuf, degacc.at[pl.ds(s * ROWS_PT, ROWS_PT)])
    plsc.subcore_barrier()

    for cp in cps:
        cp.wait()
    pltpu.sync_copy(rows, x0.at[c, s])

    @pl.loop(0, DEGC // 5)
    def _(g):
        ds_ = [pltpu.async_copy(
            ones_v, degacc.at[idxd.at[c * DEGC + 5 * g + i]], sems, add=True)
               for i in range(5)]
        for d in ds_:
            d.wait()

    plsc.subcore_barrier()

    pltpu.sync_copy(degacc.at[pl.ds(s * ROWS_PT, ROWS_PT)], degbuf)
    pltpu.sync_copy(degbuf, degp.at[c].at[pl.ds(s * ROWS_PT, ROWS_PT)])


@functools.lru_cache(maxsize=None)
def _make_sc_init():
    mesh = plsc.VectorSubcoreMesh(core_axis_name="c", subcore_axis_name="s",
                                  num_cores=NCORES, num_subcores=NSUB)
    return pl.kernel(
        _sc_init_body,
        out_type=(
            jax.ShapeDtypeStruct((NCORES, NSUB, 5, WCH, HHALF), jnp.float32),
            jax.ShapeDtypeStruct((NCORES, NPAD, 16), jnp.float32),
        ),
        mesh=mesh,
        compiler_params=_SC_PARAMS,
        scratch_types=[
            pltpu.VMEM((5, WCH), jnp.int32),
            pltpu.VMEM((5, WCH, HHALF), jnp.float32),
            pltpu.VMEM((NCHT, CHUNK), jnp.int32),
            pltpu.VMEM((CHUNK, 16), jnp.float32),
            pltpu.VMEM((ROWS_PT, 16), jnp.float32),
            pltpu.VMEM_SHARED((NPAD, 16), jnp.float32),
            pltpu.SemaphoreType.DMA,
            pltpu.SemaphoreType.DMA,
        ],
    )


def _sc_spmm_body(x_st, srcs, dsts, parts, idxs, idxd, rows, zbuf,
                  acc, semg, sems):
    c = lax.axis_index("c")
    s = lax.axis_index("s")

    pltpu.sync_copy(srcs.at[s], idxs)
    pltpu.sync_copy(dsts.at[s], idxd)

    @pl.loop(0, WCH * (HHALF // 16))
    def _(i):
        zbuf[i // (HHALF // 16), pl.ds((i % (HHALF // 16)) * 16, 16)] = (
            jnp.zeros((16,), jnp.float32))

    for k in range(ROWS_PT // WCH):
        pltpu.sync_copy(zbuf, acc.at[pl.ds(s * ROWS_PT + k * WCH, WCH)])
    plsc.subcore_barrier()

    @pl.loop(0, RND)
    def _(r):
        cps = [pltpu.async_copy(x_st.at[c].at[idxs.at[NBUF * r + b]],
                                rows.at[b], semg[b])
               for b in range(NBUF)]
        scs = []
        for b in range(NBUF):
            cps[b].wait()
            scs.append(pltpu.async_copy(
                rows.at[b], acc.at[idxd.at[NBUF * r + b]], sems[b], add=True))
        for sc in scs:
            sc.wait()

    plsc.subcore_barrier()

    for k in range(ROWS_PT // WCH):
        base = s * ROWS_PT + k * WCH
        pltpu.sync_copy(acc.at[pl.ds(base, WCH)], zbuf)
        pltpu.sync_copy(zbuf, parts.at[c].at[pl.ds(base, WCH)])


@functools.lru_cache(maxsize=None)
def _make_sc_spmm():
    mesh = plsc.VectorSubcoreMesh(core_axis_name="c", subcore_axis_name="s",
                                  num_cores=NCORES, num_subcores=NSUB)
    return pl.kernel(
        _sc_spmm_body,
        out_type=jax.ShapeDtypeStruct((NCORES, NPAD, HHALF), jnp.float32),
        mesh=mesh,
        compiler_params=_SC_PARAMS,
        scratch_types=[
            pltpu.VMEM((NCHT, CHUNK), jnp.int32),
            pltpu.VMEM((NCHT, CHUNK), jnp.int32),
            pltpu.VMEM((NBUF, CHUNK, HHALF), jnp.float32),
            pltpu.VMEM((WCH, HHALF), jnp.float32),
            pltpu.VMEM_SHARED((NPAD, HHALF), jnp.float32),
            [pltpu.SemaphoreType.DMA] * NBUF,
            [pltpu.SemaphoreType.DMA] * NBUF,
        ],
    )


BLK = 1000


def _agg_block(parts_ref, degp_ref):
    deg = degp_ref[0, :, 0] + degp_ref[1, :, 0]
    inv = 1.0 / jnp.maximum(deg, 1.0)
    agg = jnp.concatenate([parts_ref[0], parts_ref[1]], axis=1)
    return agg * inv[:, None]


def _tc_update_body(parts_ref, degp_ref, x_ref, wg_ref, wa_ref, out_ref):
    agg = _agg_block(parts_ref, degp_ref)
    g = jnp.dot(agg, wg_ref[...], preferred_element_type=jnp.float32)
    z = 1.0 / (1.0 + jnp.exp(-g))
    xb = jnp.concatenate([x_ref[0], x_ref[1]], axis=1)
    xn = xb + z * jnp.dot(agg, wa_ref[...], preferred_element_type=jnp.float32)
    out_ref[0] = xn[:, :HHALF]
    out_ref[1] = xn[:, HHALF:]


_tc_update = pl.pallas_call(
    _tc_update_body,
    out_shape=jax.ShapeDtypeStruct((NCORES, NPAD, HHALF), jnp.float32),
    grid=(N // BLK,),
    in_specs=[
        pl.BlockSpec((NCORES, BLK, HHALF), lambda i: (0, i, 0)),
        pl.BlockSpec((NCORES, BLK, 16), lambda i: (0, i, 0)),
        pl.BlockSpec((NCORES, BLK, HHALF), lambda i: (0, i, 0)),
        pl.BlockSpec((HID, HID), lambda i: (0, 0)),
        pl.BlockSpec((HID, HID), lambda i: (0, 0)),
    ],
    out_specs=pl.BlockSpec((NCORES, BLK, HHALF), lambda i: (0, i, 0)),
)


def _tc_readout_body(x_ref, w1_ref, b1_ref, w2_ref, b2_ref, out_ref):
    xb = jnp.concatenate([x_ref[0], x_ref[1]], axis=1)
    mid = jnp.maximum(
        jnp.dot(xb, w1_ref[...], preferred_element_type=jnp.float32)
        + b1_ref[...], 0.0)
    out_ref[...] = jnp.dot(
        mid, w2_ref[...], preferred_element_type=jnp.float32) + b2_ref[...]


_tc_readout = pl.pallas_call(
    _tc_readout_body,
    out_shape=jax.ShapeDtypeStruct((N, NCLS), jnp.float32),
    grid=(N // BLK,),
    in_specs=[
        pl.BlockSpec((NCORES, BLK, HHALF), lambda i: (0, i, 0)),
        pl.BlockSpec((HID, HID // 2), lambda i: (0, 0)),
        pl.BlockSpec((HID // 2,), lambda i: (0,)),
        pl.BlockSpec((HID // 2, NCLS), lambda i: (0, 0)),
        pl.BlockSpec((NCLS,), lambda i: (0,)),
    ],
    out_specs=pl.BlockSpec((BLK, NCLS), lambda i: (i, 0)),
)


def kernel(h, edge_index, e, emb_h, Wa, Wg, W1, b1, W2, b2):
    del e
    h_pad = jnp.concatenate(
        [h.astype(jnp.int32), jnp.zeros((NPAD - N,), jnp.int32)])
    h3 = h_pad.reshape(NSUB, 5, WCH)
    emb_st = jnp.stack([emb_h[:, :HHALF], emb_h[:, HHALF:]])
    srcs = edge_index[0].astype(jnp.int32).reshape(NSUB, NCHT, CHUNK)
    dsts = edge_index[1].astype(jnp.int32).reshape(NSUB, NCHT, CHUNK)

    sc_init = _make_sc_init()
    sc_spmm = _make_sc_spmm()
    x0_t, degp = sc_init(h3, emb_st, dsts)
    x_st = x0_t.reshape(NCORES, NPAD, HHALF)

    def layer(l, x_st):
        parts = sc_spmm(x_st, srcs, dsts)
        return _tc_update(parts, degp, x_st, Wg[l], Wa[l])

    x_st = lax.fori_loop(0, NLAYER, layer, x_st)
    return _tc_readout(x_st, W1, b1, W2, b2)


# speedup vs baseline: 10.1631x; 1.1739x over previous
import functools

import jax
import jax.numpy as jnp
from jax import lax
from jax.experimental import pallas as pl
from jax.experimental.pallas import tpu as pltpu
from jax.experimental.pallas import tpu_sc as plsc

N = 10000
E = 320000
HID = 128
HHALF = HID // 2
IN_DIM = 64
NCLS = 8
NLAYER = 4

NCORES = 2
NSUB = 16
EPT = E // NSUB
CHUNK = 125
NCHT = EPT // CHUNK
NBUF = 5
RND = NCHT // NBUF
DEGC = NCHT // NCORES
NPAD = 10240
ROWS_PT = NPAD // NSUB
WCH = 128

_SC_PARAMS = pltpu.CompilerParams(use_tc_tiling_on_sc=False)


def _sc_init_body(h3, emb_st, dsts, x0, degp, hbuf, rows, idxd, ones_v,
                  degbuf, degacc, semg, sems):
    c = lax.axis_index("c")
    s = lax.axis_index("s")

    pltpu.sync_copy(dsts.at[s], idxd)

    pltpu.sync_copy(h3.at[s], hbuf)
    cps = [pltpu.async_copy(emb_st.at[c].at[hbuf.at[k]], rows.at[k], semg)
           for k in range(5)]

    @pl.loop(0, CHUNK)
    def _(i):
        ones_v[i, :] = jnp.ones((16,), jnp.float32)

    @pl.loop(0, ROWS_PT)
    def _(i):
        degbuf[i, :] = jnp.zeros((16,), jnp.float32)

    pltpu.sync_copy(degbuf, degacc.at[pl.ds(s * ROWS_PT, ROWS_PT)])
    plsc.subcore_barrier()

    for cp in cps:
        cp.wait()
    pltpu.sync_copy(rows, x0.at[c, s])

    @pl.loop(0, DEGC // 5)
    def _(g):
        ds_ = [pltpu.async_copy(
            ones_v, degacc.at[idxd.at[c * DEGC + 5 * g + i]], sems, add=True)
               for i in range(5)]
        for d in ds_:
            d.wait()

    plsc.subcore_barrier()

    pltpu.sync_copy(degacc.at[pl.ds(s * ROWS_PT, ROWS_PT)], degbuf)
    pltpu.sync_copy(degbuf, degp.at[c].at[pl.ds(s * ROWS_PT, ROWS_PT)])


@functools.lru_cache(maxsize=None)
def _make_sc_init():
    mesh = plsc.VectorSubcoreMesh(core_axis_name="c", subcore_axis_name="s",
                                  num_cores=NCORES, num_subcores=NSUB)
    return pl.kernel(
        _sc_init_body,
        out_type=(
            jax.ShapeDtypeStruct((NCORES, NSUB, 5, WCH, HHALF), jnp.float32),
            jax.ShapeDtypeStruct((NCORES, NPAD, 16), jnp.float32),
        ),
        mesh=mesh,
        compiler_params=_SC_PARAMS,
        scratch_types=[
            pltpu.VMEM((5, WCH), jnp.int32),
            pltpu.VMEM((5, WCH, HHALF), jnp.float32),
            pltpu.VMEM((NCHT, CHUNK), jnp.int32),
            pltpu.VMEM((CHUNK, 16), jnp.float32),
            pltpu.VMEM((ROWS_PT, 16), jnp.float32),
            pltpu.VMEM_SHARED((NPAD, 16), jnp.float32),
            pltpu.SemaphoreType.DMA,
            pltpu.SemaphoreType.DMA,
        ],
    )


def _sc_spmm_body(x_st, srcs, dsts, parts, idxs, idxd, rows, zbuf,
                  acc, semg, sems):
    c = lax.axis_index("c")
    s = lax.axis_index("s")

    pltpu.sync_copy(srcs.at[s], idxs)
    pltpu.sync_copy(dsts.at[s], idxd)

    @pl.loop(0, WCH * (HHALF // 16))
    def _(i):
        zbuf[i // (HHALF // 16), pl.ds((i % (HHALF // 16)) * 16, 16)] = (
            jnp.zeros((16,), jnp.float32))

    for k in range(ROWS_PT // WCH):
        pltpu.sync_copy(zbuf, acc.at[pl.ds(s * ROWS_PT + k * WCH, WCH)])
    plsc.subcore_barrier()

    def _wait_gather(b):
        pltpu.make_async_copy(x_st.at[c].at[idxs.at[0]], rows.at[b],
                              semg[b]).wait()

    def _drain_scatter(b):
        pltpu.make_async_copy(rows.at[b], acc.at[idxd.at[0]], sems[b]).wait()

    for b in range(NBUF):
        pltpu.async_copy(x_st.at[c].at[idxs.at[b]], rows.at[b], semg[b])

    @pl.loop(0, RND - 1)
    def _(r):
        for b in range(NBUF):
            _wait_gather(b)
            pltpu.async_copy(rows.at[b], acc.at[idxd.at[NBUF * r + b]],
                             sems[b], add=True)
        for b in range(NBUF):
            _drain_scatter(b)
            pltpu.async_copy(x_st.at[c].at[idxs.at[NBUF * (r + 1) + b]],
                             rows.at[b], semg[b])

    for b in range(NBUF):
        _wait_gather(b)
        pltpu.async_copy(rows.at[b], acc.at[idxd.at[(RND - 1) * NBUF + b]],
                         sems[b], add=True)
    for b in range(NBUF):
        _drain_scatter(b)

    plsc.subcore_barrier()

    for k in range(ROWS_PT // WCH):
        base = s * ROWS_PT + k * WCH
        pltpu.sync_copy(acc.at[pl.ds(base, WCH)], zbuf)
        pltpu.sync_copy(zbuf, parts.at[c].at[pl.ds(base, WCH)])


@functools.lru_cache(maxsize=None)
def _make_sc_spmm():
    mesh = plsc.VectorSubcoreMesh(core_axis_name="c", subcore_axis_name="s",
                                  num_cores=NCORES, num_subcores=NSUB)
    return pl.kernel(
        _sc_spmm_body,
        out_type=jax.ShapeDtypeStruct((NCORES, NPAD, HHALF), jnp.float32),
        mesh=mesh,
        compiler_params=_SC_PARAMS,
        scratch_types=[
            pltpu.VMEM((NCHT, CHUNK), jnp.int32),
            pltpu.VMEM((NCHT, CHUNK), jnp.int32),
            pltpu.VMEM((NBUF, CHUNK, HHALF), jnp.float32),
            pltpu.VMEM((WCH, HHALF), jnp.float32),
            pltpu.VMEM_SHARED((NPAD, HHALF), jnp.float32),
            [pltpu.SemaphoreType.DMA] * NBUF,
            [pltpu.SemaphoreType.DMA] * NBUF,
        ],
    )


BLK = 2000


def _agg_block(parts_ref, degp_ref):
    deg = degp_ref[0, :, 0] + degp_ref[1, :, 0]
    inv = 1.0 / jnp.maximum(deg, 1.0)
    agg = jnp.concatenate([parts_ref[0], parts_ref[1]], axis=1)
    return agg * inv[:, None]


def _tc_update_body(parts_ref, degp_ref, x_ref, wg_ref, wa_ref, out_ref):
    agg = _agg_block(parts_ref, degp_ref)
    g = jnp.dot(agg, wg_ref[...], preferred_element_type=jnp.float32)
    z = 1.0 / (1.0 + jnp.exp(-g))
    xb = jnp.concatenate([x_ref[0], x_ref[1]], axis=1)
    xn = xb + z * jnp.dot(agg, wa_ref[...], preferred_element_type=jnp.float32)
    out_ref[0] = xn[:, :HHALF]
    out_ref[1] = xn[:, HHALF:]


_tc_update = pl.pallas_call(
    _tc_update_body,
    out_shape=jax.ShapeDtypeStruct((NCORES, NPAD, HHALF), jnp.float32),
    grid=(N // BLK,),
    in_specs=[
        pl.BlockSpec((NCORES, BLK, HHALF), lambda i: (0, i, 0)),
        pl.BlockSpec((NCORES, BLK, 16), lambda i: (0, i, 0)),
        pl.BlockSpec((NCORES, BLK, HHALF), lambda i: (0, i, 0)),
        pl.BlockSpec((HID, HID), lambda i: (0, 0)),
        pl.BlockSpec((HID, HID), lambda i: (0, 0)),
    ],
    out_specs=pl.BlockSpec((NCORES, BLK, HHALF), lambda i: (0, i, 0)),
)


def _tc_readout_body(x_ref, w1_ref, b1_ref, w2_ref, b2_ref, out_ref):
    xb = jnp.concatenate([x_ref[0], x_ref[1]], axis=1)
    mid = jnp.maximum(
        jnp.dot(xb, w1_ref[...], preferred_element_type=jnp.float32)
        + b1_ref[...], 0.0)
    out_ref[...] = jnp.dot(
        mid, w2_ref[...], preferred_element_type=jnp.float32) + b2_ref[...]


_tc_readout = pl.pallas_call(
    _tc_readout_body,
    out_shape=jax.ShapeDtypeStruct((N, NCLS), jnp.float32),
    grid=(N // BLK,),
    in_specs=[
        pl.BlockSpec((NCORES, BLK, HHALF), lambda i: (0, i, 0)),
        pl.BlockSpec((HID, HID // 2), lambda i: (0, 0)),
        pl.BlockSpec((HID // 2,), lambda i: (0,)),
        pl.BlockSpec((HID // 2, NCLS), lambda i: (0, 0)),
        pl.BlockSpec((NCLS,), lambda i: (0,)),
    ],
    out_specs=pl.BlockSpec((BLK, NCLS), lambda i: (i, 0)),
)


def kernel(h, edge_index, e, emb_h, Wa, Wg, W1, b1, W2, b2):
    del e
    h_pad = jnp.concatenate(
        [h.astype(jnp.int32), jnp.zeros((NPAD - N,), jnp.int32)])
    h3 = h_pad.reshape(NSUB, 5, WCH)
    emb_st = jnp.stack([emb_h[:, :HHALF], emb_h[:, HHALF:]])
    srcs = edge_index[0].astype(jnp.int32).reshape(NSUB, NCHT, CHUNK)
    dsts = edge_index[1].astype(jnp.int32).reshape(NSUB, NCHT, CHUNK)

    sc_init = _make_sc_init()
    sc_spmm = _make_sc_spmm()
    x0_t, degp = sc_init(h3, emb_st, dsts)
    x_st = x0_t.reshape(NCORES, NPAD, HHALF)

    def layer(l, x_st):
        parts = sc_spmm(x_st, srcs, dsts)
        return _tc_update(parts, degp, x_st, Wg[l], Wa[l])

    x_st = lax.fori_loop(0, NLAYER, layer, x_st)
    return _tc_readout(x_st, W1, b1, W2, b2)
